# baseline (speedup 1.0000x reference)
import functools

import jax
import jax.numpy as jnp
from jax import lax
from jax.experimental import pallas as pl
from jax.experimental.pallas import tpu as pltpu
from jax.experimental.pallas import tpu_sc as plsc

T = 2048
D = 2048
E = 8
F = 1024
K = 2
FS = 2048
BT = 256
GE = T * K // BT + E
PE = GE * BT
FK = 256


def _router_body(x_ref, rw_ref, a1_ref, a2_ref, w1_ref, w2_ref):
    x = x_ref[...]
    logits = lax.dot_general(x, rw_ref[...], (((1,), (1,)), ((), ())),
                             preferred_element_type=jnp.float32)
    m = jnp.max(logits, axis=1, keepdims=True)
    p = jnp.exp(logits - m)
    scores = p / jnp.sum(p, axis=1, keepdims=True)
    e_iota = lax.broadcasted_iota(jnp.int32, scores.shape, 1)
    m1 = jnp.max(scores, axis=1, keepdims=True)
    a1 = jnp.min(jnp.where(scores == m1, e_iota, E), axis=1, keepdims=True)
    s2 = jnp.where(e_iota == a1, -1.0, scores)
    m2 = jnp.max(s2, axis=1, keepdims=True)
    a2 = jnp.min(jnp.where(s2 == m2, e_iota, E), axis=1, keepdims=True)
    tot = m1 + m2
    a1_ref[...] = a1
    a2_ref[...] = a2
    w1_ref[...] = m1 / tot
    w2_ref[...] = m2 / tot


def _router(x, router_w):
    nt = T // BT
    outs = (
        jax.ShapeDtypeStruct((T, 1), jnp.int32),
        jax.ShapeDtypeStruct((T, 1), jnp.int32),
        jax.ShapeDtypeStruct((T, 1), jnp.float32),
        jax.ShapeDtypeStruct((T, 1), jnp.float32),
    )
    o_spec = pl.BlockSpec((BT, 1), lambda i: (i, 0))
    return pl.pallas_call(
        _router_body,
        grid=(nt,),
        in_specs=[pl.BlockSpec((BT, D), lambda i: (i, 0)),
                  pl.BlockSpec((E, D), lambda i: (0, 0))],
        out_specs=(o_spec, o_spec, o_spec, o_spec),
        out_shape=outs,
    )(x, router_w)


def _gmm_body(meta_ref, xs_ref, w1_ref, v1_ref, w2_ref, ys_ref):
    g = pl.program_id(0)

    @pl.when(g < meta_ref[GE])
    def _():
        x = xs_ref[...].astype(jnp.bfloat16)
        t1 = lax.dot_general(x, w1_ref[0].astype(jnp.bfloat16),
                             (((1,), (1,)), ((), ())),
                             preferred_element_type=jnp.float32)
        t2 = lax.dot_general(x, v1_ref[0].astype(jnp.bfloat16),
                             (((1,), (1,)), ((), ())),
                             preferred_element_type=jnp.float32)
        h = t1 * lax.logistic(t1) * t2
        ys_ref[...] = lax.dot_general(
            h.astype(jnp.bfloat16), w2_ref[0].astype(jnp.bfloat16),
            (((1,), (0,)), ((), ())), preferred_element_type=jnp.float32)


def _gmm(xs, w1, v1, w2, meta):
    grid_spec = pltpu.PrefetchScalarGridSpec(
        num_scalar_prefetch=1,
        grid=(GE,),
        in_specs=[
            pl.BlockSpec((BT, D), lambda g, m: (g, 0)),
            pl.BlockSpec((1, F, D), lambda g, m: (m[g], 0, 0)),
            pl.BlockSpec((1, F, D), lambda g, m: (m[g], 0, 0)),
            pl.BlockSpec((1, F, D), lambda g, m: (m[g], 0, 0)),
        ],
        out_specs=pl.BlockSpec((BT, D), lambda g, m: (g, 0)),
    )
    return pl.pallas_call(
        _gmm_body,
        grid_spec=grid_spec,
        out_shape=jax.ShapeDtypeStruct((PE, D), jnp.float32),
    )(meta, xs, w1, v1, w2)


def _shared_body(x_ref, sg_ref, su_ref, sd_ref, ysh_ref):
    c = pl.program_id(1)
    fk = pl.program_id(2)
    x = x_ref[...].astype(jnp.bfloat16)
    t1 = lax.dot_general(x, sg_ref[0].astype(jnp.bfloat16),
                         (((1,), (1,)), ((), ())),
                         preferred_element_type=jnp.float32)
    t2 = lax.dot_general(x, su_ref[0].astype(jnp.bfloat16),
                         (((1,), (1,)), ((), ())),
                         preferred_element_type=jnp.float32)
    h = t1 * lax.logistic(t1) * t2
    y = lax.dot_general(h.astype(jnp.bfloat16), sd_ref[...].astype(jnp.bfloat16),
                        (((1,), (1,)), ((), ())),
                        preferred_element_type=jnp.float32)

    @pl.when((c == 0) & (fk == 0))
    def _():
        ysh_ref[...] = y

    @pl.when((c != 0) | (fk != 0))
    def _():
        ysh_ref[...] += y


def _shared(x, shared_gate, shared_up, shared_down):
    BTS = 1024
    sg = shared_gate.reshape(FS // F, F, D)
    su = shared_up.reshape(FS // F, F, D)
    nt = T // BTS
    nc = FS // F
    nf = F // FK
    return pl.pallas_call(
        _shared_body,
        grid=(nt, nc, nf),
        in_specs=[
            pl.BlockSpec((BTS, D), lambda t, c, f: (t, 0)),
            pl.BlockSpec((1, FK, D), lambda t, c, f: (c, f, 0)),
            pl.BlockSpec((1, FK, D), lambda t, c, f: (c, f, 0)),
            pl.BlockSpec((D, FK), lambda t, c, f: (0, c * (F // FK) + f)),
        ],
        out_specs=pl.BlockSpec((BTS, D), lambda t, c, f: (t, 0)),
        out_shape=jax.ShapeDtypeStruct((T, D), jnp.float32),
    )(x, sg, su, shared_down)


NW = 32
GCH = 16


def _sc_gather_body(x_hbm, idx_hbm, out_hbm, idx_v, buf0, buf1, sem0, sem1):
    rows = PE // NW
    wid = lax.axis_index("s") * 2 + lax.axis_index("c")
    base = wid * rows
    pltpu.sync_copy(idx_hbm.at[pl.ds(base, rows)], idx_v)
    prev = None
    for c in range(rows // GCH):
        buf, sem = (buf0, sem0) if c % 2 == 0 else (buf1, sem1)
        cp = pltpu.async_copy(x_hbm.at[idx_v.at[pl.ds(c * GCH, GCH)]], buf,
                              sem)
        if prev is not None:
            pcp, pbuf, pc = prev
            pcp.wait()
            pltpu.sync_copy(pbuf, out_hbm.at[pl.ds(base + pc * GCH, GCH)])
        prev = (cp, buf, c)
    pcp, pbuf, pc = prev
    pcp.wait()
    pltpu.sync_copy(pbuf, out_hbm.at[pl.ds(base + pc * GCH, GCH)])


def _sc_gather(x, src_tok):
    mesh = plsc.VectorSubcoreMesh(core_axis_name="c", subcore_axis_name="s")
    k = functools.partial(
        pl.kernel,
        out_type=jax.ShapeDtypeStruct((PE, D), jnp.float32),
        mesh=mesh,
        scratch_types=[
            pltpu.VMEM((PE // NW,), jnp.int32),
            pltpu.VMEM((GCH, D), jnp.float32),
            pltpu.VMEM((GCH, D), jnp.float32),
            pltpu.SemaphoreType.DMA,
            pltpu.SemaphoreType.DMA,
        ],
    )(_sc_gather_body)
    return k(x, src_tok)


CCH = 8


def _sc_combine_body(ys_hbm, ysh_hbm, p1_hbm, p2_hbm, w1_hbm, w2_hbm,
                     out_hbm, p1_v, p2_v, w1_v, w2_v, a_v, b_v, s_v, o_v,
                     sema, semb, sems):
    toks = T // NW
    wid = lax.axis_index("s") * 2 + lax.axis_index("c")
    base = wid * toks
    pltpu.sync_copy(p1_hbm.at[pl.ds(base, toks)], p1_v)
    pltpu.sync_copy(p2_hbm.at[pl.ds(base, toks)], p2_v)
    pltpu.sync_copy(w1_hbm.at[pl.ds(base, toks)], w1_v)
    pltpu.sync_copy(w2_hbm.at[pl.ds(base, toks)], w2_v)
    nv = D // 16
    for c in range(toks // CCH):
        cpa = pltpu.async_copy(ys_hbm.at[p1_v.at[pl.ds(c * CCH, CCH)]], a_v,
                               sema)
        cpb = pltpu.async_copy(ys_hbm.at[p2_v.at[pl.ds(c * CCH, CCH)]], b_v,
                               semb)
        cps = pltpu.async_copy(ysh_hbm.at[pl.ds(base + c * CCH, CCH)], s_v,
                               sems)
        cpa.wait()
        cpb.wait()
        cps.wait()
        wav = w1_v[pl.ds((c // 2) * 16, 16)]
        wbv = w2_v[pl.ds((c // 2) * 16, 16)]
        for r in range(CCH):
            wa = wav[(c % 2) * CCH + r]
            wb = wbv[(c % 2) * CCH + r]

            def vec_body(v, _):
                sl = pl.ds(v * 16, 16)
                o_v[r, sl] = (a_v[r, sl] * wa + b_v[r, sl] * wb
                              + s_v[r, sl])
                return ()

            lax.fori_loop(0, nv, vec_body, (), unroll=4)
        pltpu.sync_copy(o_v, out_hbm.at[pl.ds(base + c * CCH, CCH)])


def _sc_combine(ys, ysh, p1, p2, w1, w2):
    mesh = plsc.VectorSubcoreMesh(core_axis_name="c", subcore_axis_name="s")
    toks = T // NW
    k = functools.partial(
        pl.kernel,
        out_type=jax.ShapeDtypeStruct((T, D), jnp.float32),
        mesh=mesh,
        scratch_types=[
            pltpu.VMEM((toks,), jnp.int32),
            pltpu.VMEM((toks,), jnp.int32),
            pltpu.VMEM((toks,), jnp.float32),
            pltpu.VMEM((toks,), jnp.float32),
            pltpu.VMEM((CCH, D), jnp.float32),
            pltpu.VMEM((CCH, D), jnp.float32),
            pltpu.VMEM((CCH, D), jnp.float32),
            pltpu.VMEM((CCH, D), jnp.float32),
            pltpu.SemaphoreType.DMA,
            pltpu.SemaphoreType.DMA,
            pltpu.SemaphoreType.DMA,
        ],
    )(_sc_combine_body)
    return k(ys, ysh, p1, p2, w1, w2)


def kernel(hidden_states, router_w, w1, v1, w2, shared_gate, shared_up,
           shared_down):
    x = hidden_states.reshape(T, D)

    a1, a2, wt1, wt2 = _router(x, router_w)
    a1 = a1[:, 0]
    a2 = a2[:, 0]
    wt1 = wt1[:, 0]
    wt2 = wt2[:, 0]

    ee = jnp.stack([a1, a2], axis=1).reshape(-1)
    onehot = (ee[:, None] == jnp.arange(E)[None, :]).astype(jnp.int32)
    counts = jnp.sum(onehot, axis=0)
    rank = jnp.take_along_axis(jnp.cumsum(onehot, axis=0) - onehot,
                               ee[:, None], axis=1)[:, 0]
    nblk = (counts + BT - 1) // BT
    blk_start = jnp.concatenate([jnp.zeros((1,), jnp.int32),
                                 jnp.cumsum(nblk).astype(jnp.int32)])
    off = blk_start[:E] * BT
    pos = jnp.take(off, ee) + rank
    src_tok = jnp.zeros((PE,), jnp.int32).at[pos].set(
        jnp.arange(T * K, dtype=jnp.int32) // K)
    g_iota = jnp.arange(GE, dtype=jnp.int32)
    block_expert = jnp.clip(
        jnp.sum((g_iota[:, None] >= blk_start[None, :E]).astype(jnp.int32),
                axis=1) - 1, 0, E - 1)
    meta = jnp.concatenate([block_expert, blk_start[E:]])
    posk = pos.reshape(T, K)
    p1 = posk[:, 0]
    p2 = posk[:, 1]

    xs = _sc_gather(x, src_tok)

    ys = _gmm(xs, w1, v1, w2, meta)
    ysh = _shared(x, shared_gate, shared_up, shared_down)

    out = _sc_combine(ys, ysh, p1, p2, wt1, wt2)
    return out.reshape(1, T, D)
